# Initial kernel scaffold; baseline (speedup 1.0000x reference)
import jax
import jax.numpy as jnp
from jax.experimental import pallas as pl


def kernel(x, edge_index, edge_attr, We1, be1, We2, be2, Wn1, bn1, Wn2, bn2):
    raise NotImplementedError("write your pallas kernel here")


# speedup vs baseline: 2.8858x; 2.8858x over previous
import functools

import jax
import jax.numpy as jnp
from jax import lax
from jax.experimental import pallas as pl
from jax.experimental.pallas import tpu as pltpu
from jax.experimental.pallas import tpu_sc as plsc

_NC = 2
_NS = 16
_NW = _NC * _NS
_CB = 128


def _gather_sum(a, b, src, dst, *, E, D):
    R = E // _CB
    T = (R + _NW - 1) // _NW
    mesh = plsc.VectorSubcoreMesh(core_axis_name="c", subcore_axis_name="s")

    def body(a_hbm, b_hbm, src_hbm, dst_hbm, g_hbm,
             idx_s, idx_d, bufa, bufb, sis, sid, sga, sgb):
        c = lax.axis_index("c")
        s = lax.axis_index("s")
        w = s * _NC + c

        def issue_idx(t, p):
            @pl.when(w + t * _NW < R)
            def _():
                r = w + t * _NW
                pltpu.async_copy(src_hbm.at[pl.ds(r * _CB, _CB)],
                                 idx_s.at[p], sis.at[p])
                pltpu.async_copy(dst_hbm.at[pl.ds(r * _CB, _CB)],
                                 idx_d.at[p], sid.at[p])

        def wait_idx(t, p):
            @pl.when(w + t * _NW < R)
            def _():
                r = w + t * _NW
                pltpu.make_async_copy(src_hbm.at[pl.ds(r * _CB, _CB)],
                                      idx_s.at[p], sis.at[p]).wait()
                pltpu.make_async_copy(dst_hbm.at[pl.ds(r * _CB, _CB)],
                                      idx_d.at[p], sid.at[p]).wait()

        def issue_gather(t, p):
            @pl.when(w + t * _NW < R)
            def _():
                pltpu.async_copy(a_hbm.at[idx_s.at[p]], bufa.at[p], sga.at[p])
                pltpu.async_copy(b_hbm.at[idx_d.at[p]], bufb.at[p], sgb.at[p])

        def wait_gather(t, p):
            @pl.when(w + t * _NW < R)
            def _():
                pltpu.make_async_copy(a_hbm.at[idx_s.at[p]], bufa.at[p],
                                      sga.at[p]).wait()
                pltpu.make_async_copy(b_hbm.at[idx_d.at[p]], bufb.at[p],
                                      sgb.at[p]).wait()

        def add_store(t, p):
            @pl.when(w + t * _NW < R)
            def _():
                r = w + t * _NW

                @pl.loop(0, _CB)
                def _row(i):
                    for q in range(D // 16):
                        sl = pl.ds(q * 16, 16)
                        bufa[p, i, sl] = bufa[p, i, sl] + bufb[p, i, sl]

                pltpu.sync_copy(bufa.at[p], g_hbm.at[pl.ds(r * _CB, _CB)])

        issue_idx(0, 0)
        wait_idx(0, 0)
        issue_gather(0, 0)
        issue_idx(1, 1)

        @pl.loop(0, T // 2)
        def _pair(tt):
            for p in (0, 1):
                t = tt * 2 + p
                wait_idx(t + 1, 1 - p)
                issue_gather(t + 1, 1 - p)
                wait_gather(t, p)
                issue_idx(t + 2, p)
                add_store(t, p)

    call = pl.kernel(
        body,
        out_type=jax.ShapeDtypeStruct((E, D), jnp.float32),
        mesh=mesh,
        scratch_types=[
            pltpu.VMEM((2, _CB), jnp.int32),
            pltpu.VMEM((2, _CB), jnp.int32),
            pltpu.VMEM((2, _CB, D), jnp.float32),
            pltpu.VMEM((2, _CB, D), jnp.float32),
            pltpu.SemaphoreType.DMA((2,)),
            pltpu.SemaphoreType.DMA((2,)),
            pltpu.SemaphoreType.DMA((2,)),
            pltpu.SemaphoreType.DMA((2,)),
        ],
    )
    return call(a, b, src, dst)


_NLO = 5000
_NPH = 5240


def _seg_kernel(vals, dst, *, E, DE, with_vals):
    R = E // _CB
    T = (R + _NS - 1) // _NS
    if T % 2:
        T += 1
    mesh = plsc.VectorSubcoreMesh(core_axis_name="c", subcore_axis_name="s")

    def body(*refs):
        if with_vals:
            vals_hbm, dst_hbm, out_hbm, idx_d, buf, acc, sii, siv = refs
        else:
            dst_hbm, out_hbm, idx_d, buf, acc, sii = refs
        c = lax.axis_index("c")
        s = lax.axis_index("s")
        base = c * _NLO
        ub = jnp.where(c == 0, jnp.int32(_NLO), jnp.int32(2 ** 30))
        iota16 = lax.iota(jnp.int32, 16)
        ones = jnp.ones((16,), jnp.float32)

        @pl.loop(0, _NPH * DE // 16)
        def _zero(i):
            acc[pl.ds(i * 16, 16)] = jnp.zeros((16,), jnp.float32)

        def issue(t, p):
            @pl.when(s + t * _NS < R)
            def _():
                r = s + t * _NS
                pltpu.async_copy(dst_hbm.at[pl.ds(r * _CB, _CB)],
                                 idx_d.at[p], sii.at[p])
                if with_vals:
                    pltpu.async_copy(vals_hbm.at[pl.ds(r * _CB, _CB)],
                                     buf.at[p], siv.at[p])

        def wait(t, p):
            @pl.when(s + t * _NS < R)
            def _():
                r = s + t * _NS
                pltpu.make_async_copy(dst_hbm.at[pl.ds(r * _CB, _CB)],
                                      idx_d.at[p], sii.at[p]).wait()
                if with_vals:
                    pltpu.make_async_copy(vals_hbm.at[pl.ds(r * _CB, _CB)],
                                          buf.at[p], siv.at[p]).wait()

        def process(t, p):
            @pl.when(s + t * _NS < R)
            def _():
                @pl.loop(0, _CB // 16)
                def _grp(g):
                    for j in range(16):
                        e = g * 16 + j
                        dj = plsc.load_gather(
                            idx_d, [jnp.full((16,), p, jnp.int32),
                                    jnp.full((16,), e, jnp.int32)])
                        m = (dj >= base) & (dj < ub)
                        addr = jnp.where(m, dj - base, 0) * DE + iota16
                        v = buf[p, e, :] if with_vals else ones
                        plsc.addupdate_scatter(acc, [addr], v, mask=m)

        issue(0, 0)
        issue(1, 1)

        @pl.loop(0, T // 2)
        def _pair(tt):
            for p in (0, 1):
                t = tt * 2 + p
                wait(t, p)
                process(t, p)
                issue(t + 2, p)

        pltpu.sync_copy(acc, out_hbm.at[c, s])

    scratch = [
        pltpu.VMEM((2, _CB), jnp.int32),
        pltpu.VMEM((2, _CB, DE), jnp.float32),
        pltpu.VMEM((_NPH * DE,), jnp.float32),
        pltpu.SemaphoreType.DMA((2,)),
    ]
    if with_vals:
        scratch.append(pltpu.SemaphoreType.DMA((2,)))
    call = pl.kernel(
        body,
        out_type=jax.ShapeDtypeStruct((_NC, _NS, _NPH * DE), jnp.float32),
        mesh=mesh,
        scratch_types=scratch,
        compiler_params=pltpu.CompilerParams(needs_layout_passes=False),
    )
    out = call(vals, dst) if with_vals else call(dst)
    return out.reshape(_NC, _NS, _NPH, DE)


def _segment_sum(vals, dst, *, E, DE):
    return _seg_kernel(vals, dst, E=E, DE=DE, with_vals=True)


def _segment_count(dst, *, E, DE):
    return _seg_kernel(None, dst, E=E, DE=DE, with_vals=False)


def _proj_body(x_ref, w1a_ref, w1b_ref, be1_ref, a_ref, b_ref):
    x = x_ref[...]
    a_ref[...] = jnp.dot(x, w1a_ref[...],
                         preferred_element_type=jnp.float32) + be1_ref[...]
    b_ref[...] = jnp.dot(x, w1b_ref[...], preferred_element_type=jnp.float32)


def _project(x, w1a, w1b, be1, *, N, D, H, BN=1000):
    grid = (N // BN,)
    return pl.pallas_call(
        _proj_body,
        grid=grid,
        in_specs=[
            pl.BlockSpec((BN, D), lambda i: (i, 0)),
            pl.BlockSpec((D, H), lambda i: (0, 0)),
            pl.BlockSpec((D, H), lambda i: (0, 0)),
            pl.BlockSpec((1, H), lambda i: (0, 0)),
        ],
        out_specs=[
            pl.BlockSpec((BN, H), lambda i: (i, 0)),
            pl.BlockSpec((BN, H), lambda i: (i, 0)),
        ],
        out_shape=[
            jax.ShapeDtypeStruct((N, H), jnp.float32),
            jax.ShapeDtypeStruct((N, H), jnp.float32),
        ],
    )(x, w1a, w1b, be1)


def _edge_body(g_ref, ea_ref, w1c_ref, we2_ref, be2_ref, he_ref, ean_ref):
    ea = ea_ref[...]
    h = g_ref[...] + jnp.dot(ea, w1c_ref[...],
                             preferred_element_type=jnp.float32)
    h = jnp.maximum(h, 0.0)
    he = jnp.dot(h, we2_ref[...],
                 preferred_element_type=jnp.float32) + be2_ref[...]
    he_ref[...] = he
    ean_ref[...] = (ea + jnp.maximum(he, 0.0)) * 0.5


def _edge_mlp(g, ea, w1c, we2, be2, *, E, D, DE, H, BE=2000):
    grid = (E // BE,)
    return pl.pallas_call(
        _edge_body,
        grid=grid,
        in_specs=[
            pl.BlockSpec((BE, H), lambda i: (i, 0)),
            pl.BlockSpec((BE, DE), lambda i: (i, 0)),
            pl.BlockSpec((DE, H), lambda i: (0, 0)),
            pl.BlockSpec((H, DE), lambda i: (0, 0)),
            pl.BlockSpec((1, DE), lambda i: (0, 0)),
        ],
        out_specs=[
            pl.BlockSpec((BE, DE), lambda i: (i, 0)),
            pl.BlockSpec((BE, DE), lambda i: (i, 0)),
        ],
        out_shape=[
            jax.ShapeDtypeStruct((E, DE), jnp.float32),
            jax.ShapeDtypeStruct((E, DE), jnp.float32),
        ],
    )(g, ea, w1c, we2, be2)


def _node_body(x_ref, aggp_ref, cntp_ref, wn1x_ref, wn1a_ref, bn1_ref,
               wn2_ref, bn2_ref, w1a_ref, w1b_ref, be1_ref,
               xn_ref, a_ref, b_ref):
    x = x_ref[...]
    cnt = jnp.maximum(jnp.sum(cntp_ref[0], axis=0), 1.0)
    agg = jnp.sum(aggp_ref[0], axis=0) / cnt
    h = jnp.dot(x, wn1x_ref[...], preferred_element_type=jnp.float32)
    h = h + jnp.dot(agg, wn1a_ref[...], preferred_element_type=jnp.float32)
    h = jnp.maximum(h + bn1_ref[...], 0.0)
    hx = jnp.dot(h, wn2_ref[...],
                 preferred_element_type=jnp.float32) + bn2_ref[...]
    xn = (x + jnp.maximum(hx, 0.0)) * 0.5
    xn_ref[...] = xn
    a_ref[...] = jnp.dot(xn, w1a_ref[...],
                         preferred_element_type=jnp.float32) + be1_ref[...]
    b_ref[...] = jnp.dot(xn, w1b_ref[...], preferred_element_type=jnp.float32)


def _node_mlp(x, aggp, cntp, wn1x, wn1a, bn1, wn2, bn2, w1a, w1b, be1,
              *, N, D, DE, H, BN=1000):
    grid = (N // BN,)
    nlo = _NLO // BN
    part_spec = pl.BlockSpec((1, _NS, BN, DE),
                             lambda i: (i // nlo, 0, i % nlo, 0))
    return pl.pallas_call(
        _node_body,
        grid=grid,
        in_specs=[
            pl.BlockSpec((BN, D), lambda i: (i, 0)),
            part_spec,
            part_spec,
            pl.BlockSpec((D, H), lambda i: (0, 0)),
            pl.BlockSpec((DE, H), lambda i: (0, 0)),
            pl.BlockSpec((1, H), lambda i: (0, 0)),
            pl.BlockSpec((H, D), lambda i: (0, 0)),
            pl.BlockSpec((1, D), lambda i: (0, 0)),
            pl.BlockSpec((D, H), lambda i: (0, 0)),
            pl.BlockSpec((D, H), lambda i: (0, 0)),
            pl.BlockSpec((1, H), lambda i: (0, 0)),
        ],
        out_specs=[
            pl.BlockSpec((BN, D), lambda i: (i, 0)),
            pl.BlockSpec((BN, H), lambda i: (i, 0)),
            pl.BlockSpec((BN, H), lambda i: (i, 0)),
        ],
        out_shape=[
            jax.ShapeDtypeStruct((N, D), jnp.float32),
            jax.ShapeDtypeStruct((N, H), jnp.float32),
            jax.ShapeDtypeStruct((N, H), jnp.float32),
        ],
    )(x, aggp, cntp, wn1x, wn1a, bn1, wn2, bn2, w1a, w1b, be1)


def kernel(x, edge_index, edge_attr, We1, be1, We2, be2, Wn1, bn1, Wn2, bn2):
    N, D = x.shape
    E = edge_index.shape[1]
    DE = edge_attr.shape[1]
    NBk = We1.shape[0]
    H = We1.shape[2]

    src = edge_index[0]
    dst = edge_index[1]

    cntp = _segment_count(dst, E=E, DE=DE)
    a, b = _project(x, We1[0, :D], We1[0, D:2 * D], be1[0].reshape(1, H),
                    N=N, D=D, H=H)
    ea = edge_attr
    for k in range(NBk):
        g = _gather_sum(a, b, src, dst, E=E, D=H)
        he, ea = _edge_mlp(g, ea, We1[k, 2 * D:], We2[k],
                           be2[k].reshape(1, DE), E=E, D=D, DE=DE, H=H)
        aggp = _segment_sum(he, dst, E=E, DE=DE)
        kn = (k + 1) % NBk
        x, a, b = _node_mlp(
            x, aggp, cntp, Wn1[k, :D], Wn1[k, D:], bn1[k].reshape(1, H),
            Wn2[k], bn2[k].reshape(1, D), We1[kn, :D], We1[kn, D:2 * D],
            be1[kn].reshape(1, H), N=N, D=D, DE=DE, H=H)
    return (x, edge_index, ea)
